# baseline (speedup 1.0000x reference)
import functools

import jax
import jax.numpy as jnp
from jax import lax
from jax.experimental import pallas as pl
from jax.experimental.pallas import tpu as pltpu

H = 64


def _silu(v):
    return v * jax.nn.sigmoid(v)


def _embed_body(x_ref, w_ref, b_ref, o_ref):
    o_ref[...] = jnp.dot(x_ref[...], w_ref[...],
                         preferred_element_type=jnp.float32) + b_ref[...]


def _embed(x, w, b, block):
    n, k = x.shape
    grid = n // block
    return pl.pallas_call(
        _embed_body,
        grid=(grid,),
        in_specs=[
            pl.BlockSpec((block, k), lambda i: (i, 0)),
            pl.BlockSpec((k, H), lambda i: (0, 0)),
            pl.BlockSpec((1, H), lambda i: (0, 0)),
        ],
        out_specs=pl.BlockSpec((block, H), lambda i: (i, 0)),
        out_shape=jax.ShapeDtypeStruct((n, H), jnp.float32),
    )(x, w, b.reshape(1, H))


def _msg_body(xi_ref, xj_ref, ea_ref, w_ref, o_ref):
    w = w_ref[...]
    xi = xi_ref[...]
    xj = xj_ref[...]
    ea = ea_ref[...]
    dot = lambda a, b: jnp.dot(a, b, preferred_element_type=jnp.float32)
    g1 = _silu(dot(xi, w[0]) + dot(xj, w[1]) + dot(ea, w[2]) + w[9][0:1])
    gate = jax.nn.sigmoid(dot(g1, w[3][:, 0:1]) + w[9][1, 0])
    m1 = _silu(dot(xi, w[4]) + dot(xj, w[5]) + dot(ea, w[6]) + w[9][2:3])
    m2 = _silu(dot(m1, w[7]) + w[9][3:4])
    msg = dot(m2, w[8]) + w[9][4:5]
    o_ref[...] = gate * msg


def _msg_mlp(xi, xj, ea, cp, block):
    e = xi.shape[0]
    gw = cp["gate_net"]
    nw = cp["node_net"]
    w1 = gw[0]["W"]
    n1 = nw[0]["W"]
    wstack = jnp.stack([
        w1[0:H], w1[H:2 * H], w1[2 * H:3 * H],
        jnp.pad(gw[1]["W"], ((0, 0), (0, H - 1))),
        n1[0:H], n1[H:2 * H], n1[2 * H:3 * H],
        nw[1]["W"], nw[2]["W"],
        jnp.concatenate([
            gw[0]["b"][None],
            jnp.pad(gw[1]["b"], (0, H - 1))[None],
            nw[0]["b"][None], nw[1]["b"][None], nw[2]["b"][None],
            jnp.zeros((H - 5, H), jnp.float32),
        ], axis=0),
    ])
    grid = e // block
    return pl.pallas_call(
        _msg_body,
        grid=(grid,),
        in_specs=[
            pl.BlockSpec((block, H), lambda i: (i, 0)),
            pl.BlockSpec((block, H), lambda i: (i, 0)),
            pl.BlockSpec((block, H), lambda i: (i, 0)),
            pl.BlockSpec((10, H, H), lambda i: (0, 0, 0)),
        ],
        out_specs=pl.BlockSpec((block, H), lambda i: (i, 0)),
        out_shape=jax.ShapeDtypeStruct((e, H), jnp.float32),
    )(xi, xj, ea, wstack)


def _eupd_body(xi_ref, xj_ref, ea_ref, w_ref, o_ref):
    w = w_ref[...]
    dot = lambda a, b: jnp.dot(a, b, preferred_element_type=jnp.float32)
    ea = ea_ref[...]
    h = _silu(dot(xi_ref[...], w[0]) + dot(xj_ref[...], w[1]) +
              dot(ea, w[2]) + w[4][0:1])
    o_ref[...] = ea + dot(h, w[3]) + w[4][1:2]


def _edge_update(xi, xj, ea, ew, block):
    e = xi.shape[0]
    w1 = ew[0]["W"]
    wstack = jnp.stack([
        w1[0:H], w1[H:2 * H], w1[2 * H:3 * H], ew[1]["W"],
        jnp.concatenate([ew[0]["b"][None], ew[1]["b"][None],
                         jnp.zeros((H - 2, H), jnp.float32)], axis=0),
    ])
    grid = e // block
    return pl.pallas_call(
        _eupd_body,
        grid=(grid,),
        in_specs=[
            pl.BlockSpec((block, H), lambda i: (i, 0)),
            pl.BlockSpec((block, H), lambda i: (i, 0)),
            pl.BlockSpec((block, H), lambda i: (i, 0)),
            pl.BlockSpec((5, H, H), lambda i: (0, 0, 0)),
        ],
        out_specs=pl.BlockSpec((block, H), lambda i: (i, 0)),
        out_shape=jax.ShapeDtypeStruct((e, H), jnp.float32),
    )(xi, xj, ea, wstack)


def _skip_body(xn_ref, xs_ref, w_ref, b_ref, o_ref):
    o_ref[...] = xn_ref[...] + jnp.dot(
        xs_ref[...], w_ref[...], preferred_element_type=jnp.float32) + b_ref[...]


def _skip_add(x_new, x_skip, sp):
    n = x_new.shape[0]
    return pl.pallas_call(
        _skip_body,
        out_shape=jax.ShapeDtypeStruct((n, H), jnp.float32),
    )(x_new, x_skip, sp["W"], sp["b"].reshape(1, H))


def _readout_body(x_ref, u_ref, w_ref, o_ref):
    w = w_ref[...]
    dot = lambda a, b: jnp.dot(a, b, preferred_element_type=jnp.float32)
    pool = jnp.sum(x_ref[...], axis=0, keepdims=True) * (1.0 / x_ref.shape[0])
    u_emb = dot(u_ref[...], w[0][:6]) + w[5][0:1]
    h = _silu(dot(pool, w[1]) + dot(u_emb, w[2]) + w[5][1:2])
    h = _silu(dot(h, w[3]) + w[5][2:3])
    h = _silu(dot(h, w[4][:, 0:32]) + w[5][3:4, 0:32])
    out = dot(h, w[6][0:32, 0:1]) + w[5][4, 32]
    o_ref[...] = out


def _readout(x_final, u, params):
    f1 = params["ffn"][0]["W"]
    o1 = params["out"][0]["W"]
    o2 = params["out"][1]["W"]
    ge = params["global_embed"]
    biases = jnp.concatenate([
        ge["b"][None], params["ffn"][0]["b"][None], params["ffn"][1]["b"][None],
        jnp.pad(params["out"][0]["b"], (0, H // 2))[None],
        jnp.concatenate([jnp.zeros((32,), jnp.float32),
                         jnp.pad(params["out"][1]["b"], (0, 31))])[None],
    ], axis=0)
    biases = jnp.pad(biases, ((0, H - 5), (0, 0)))
    wstack = jnp.stack([
        jnp.pad(ge["W"], ((0, H - 6), (0, 0))),
        f1[0:H], f1[H:2 * H],
        params["ffn"][1]["W"],
        jnp.pad(o1, ((0, 0), (0, H // 2))),
        biases,
        jnp.pad(o2, ((0, 32), (0, H - 1))),
    ])
    return pl.pallas_call(
        _readout_body,
        out_shape=jax.ShapeDtypeStruct((1, 1), jnp.float32),
    )(x_final, u, wstack)


def _gather2(table, idx_a, idx_b):
    return table[idx_a], table[idx_b]


def _segsum(rows, idx, base):
    return base + jax.ops.segment_sum(rows, idx, num_segments=base.shape[0])


def kernel(x, edge_index, edge_attr, line_graph_edge_index,
           line_graph_edge_attr, u, batch, params):
    del batch
    row = edge_index[0].astype(jnp.int32)
    col = edge_index[1].astype(jnp.int32)
    lrow = line_graph_edge_index[0].astype(jnp.int32)
    lcol = line_graph_edge_index[1].astype(jnp.int32)

    xh = _embed(x, params["node_embed"]["W"], params["node_embed"]["b"], 2000)
    ea = _embed(edge_attr, params["edge_embed"]["W"],
                params["edge_embed"]["b"], 2000)
    lea = _embed(line_graph_edge_attr, params["line_edge_embed"]["W"],
                 params["line_edge_embed"]["b"], 2000)

    x_skip = xh
    for lp in params["layers"]:
        xi, xj = _gather2(xh, col, row)
        gm = _msg_mlp(xi, xj, ea, lp["atom"], 2000)
        x_new = _segsum(gm, col, xh)
        xr, xc = _gather2(x_new, row, col)
        ea_new = _edge_update(xr, xc, ea, lp["atom"]["edge_net"], 2000)
        eai, eaj = _gather2(ea_new, lcol, lrow)
        gml = _msg_mlp(eai, eaj, lea, lp["edge"], 2000)
        ea = _segsum(gml, lcol, ea_new)
        xh = _skip_add(x_new, x_skip, lp["skip"])
        x_skip = xh

    return _readout(xh, u, params)
